# baseline (speedup 1.0000x reference)
import jax
import jax.numpy as jnp
from jax.experimental import pallas as pl
from jax.experimental.pallas import tpu as pltpu

N = 10000
D_IN = 256
D_OUT = 256
BM = 400
NBIG = 24
BS = 80
NSMALL = 5
NSTEP = NBIG + NSMALL
DEPTH = 2


def _fused_kernel(adj_hbm, x_ref, w_ref, b_ref, out_ref, ring, y_ref, sems):
    i = pl.program_id(0)

    def start_copy(blk):
        slot = jax.lax.rem(blk, DEPTH)

        @pl.when(blk < NBIG)
        def _big():
            pltpu.make_async_copy(
                adj_hbm.at[pl.ds(blk * BM, BM), :],
                ring.at[slot],
                sems.at[slot],
            ).start()

        @pl.when((blk >= NBIG) & (blk < NSTEP))
        def _small():
            pltpu.make_async_copy(
                adj_hbm.at[pl.ds(NBIG * BM + (blk - NBIG) * BS, BS), :],
                ring.at[slot, pl.ds(0, BS), :],
                sems.at[slot],
            ).start()

    @pl.when(i == 0)
    def _prologue():
        start_copy(0)
        xb = x_ref[...].astype(jnp.bfloat16)
        wb = w_ref[...].astype(jnp.bfloat16)
        y = jnp.dot(xb, wb.T, preferred_element_type=jnp.float32)
        y_ref[...] = y.astype(jnp.bfloat16)

    start_copy(i + 1)

    slot = jax.lax.rem(i, DEPTH)

    @pl.when(i < NBIG)
    def _compute_big():
        pltpu.make_async_copy(
            adj_hbm.at[pl.ds(i * BM, BM), :],
            ring.at[slot],
            sems.at[slot],
        ).wait()
        a = ring[slot].astype(jnp.bfloat16)
        acc = jnp.dot(a, y_ref[...], preferred_element_type=jnp.float32)
        out_ref[...] = jnp.maximum(acc + b_ref[...], 0.0)

    @pl.when(i >= NBIG)
    def _compute_small():
        pltpu.make_async_copy(
            adj_hbm.at[pl.ds(NBIG * BM + (i - NBIG) * BS, BS), :],
            ring.at[slot, pl.ds(0, BS), :],
            sems.at[slot],
        ).wait()
        a = ring[slot, pl.ds(0, BS), :].astype(jnp.bfloat16)
        acc = jnp.dot(a, y_ref[...], preferred_element_type=jnp.float32)
        row = (i - NBIG) * BS
        out_ref[pl.ds(row, BS), :] = jnp.maximum(acc + b_ref[...], 0.0)


def _out_index(i):
    return (jnp.minimum(i, NBIG), 0)


def kernel(adj, x, W, b):
    b2 = b.reshape(1, D_OUT)
    return pl.pallas_call(
        _fused_kernel,
        grid=(NSTEP,),
        in_specs=[
            pl.BlockSpec(memory_space=pl.ANY),
            pl.BlockSpec((N, D_IN), lambda i: (0, 0)),
            pl.BlockSpec((D_OUT, D_IN), lambda i: (0, 0)),
            pl.BlockSpec((1, D_OUT), lambda i: (0, 0)),
        ],
        out_specs=pl.BlockSpec((BM, D_OUT), _out_index),
        out_shape=jax.ShapeDtypeStruct((N, D_OUT), jnp.float32),
        scratch_shapes=[
            pltpu.VMEM((DEPTH, BM, N), jnp.float32),
            pltpu.VMEM((N, D_OUT), jnp.bfloat16),
            pltpu.SemaphoreType.DMA((DEPTH,)),
        ],
    )(adj, x, W, b2)
